# baseline (speedup 1.0000x reference)
import functools

import jax
import jax.numpy as jnp
from jax import lax
from jax.experimental import pallas as pl
from jax.experimental.pallas import tpu as pltpu
from jax.experimental.pallas import tpu_sc as plsc

N = 10000
E = 320000
D = 128
NC = 2
NS = 16
NW = NC * NS
K = 128
E_PAD = 327680
N_ACC = 10240
TPB = N_ACC // NW
SCH = 2048
FLUSH = 2048
CAP = FLUSH + SCH
REG = E_PAD + FLUSH
NCH = E_PAD // SCH


def _wid():
    return lax.axis_index("c") * NS + lax.axis_index("s")


def _sc_deg_body(llist_hbm, cnt_hbm, deg_hbm, lst, cbuf, deg_v):
    t = _wid()
    lane = lax.iota(jnp.int32, 16)
    ones = jnp.ones((16,), jnp.int32)

    def zero(i, _):
        deg_v[i, :] = jnp.zeros((16,), jnp.int32)
        return jnp.int32(0)

    lax.fori_loop(jnp.int32(0), jnp.int32(TPB + 1), zero, jnp.int32(0),
                  unroll=False)
    pltpu.sync_copy(cnt_hbm.at[t], cbuf)
    total = cbuf[pl.ds(0, 16)][0]
    nch = (total + jnp.int32(SCH - 1)) // jnp.int32(SCH)

    def chunk(ci, _):
        base = t * jnp.int32(REG) + jnp.int32(ci) * jnp.int32(SCH)
        pltpu.sync_copy(llist_hbm.at[pl.ds(base, SCH)], lst)
        gbase = jnp.int32(ci) * jnp.int32(SCH)

        def gbody(g, _):
            o = jnp.int32(g) * 16
            valid = (gbase + o + lane) < total
            dv = jnp.where(valid, jnp.clip(lst[pl.ds(o, 16)], 0,
                                           jnp.int32(TPB)), jnp.int32(TPB))
            for l in range(16):
                plsc.addupdate(deg_v.at[dv[l]], ones)
            return jnp.int32(0)

        lax.fori_loop(jnp.int32(0), jnp.int32(SCH // 16), gbody, jnp.int32(0),
                      unroll=False)
        return jnp.int32(0)

    lax.fori_loop(jnp.int32(0), nch, chunk, jnp.int32(0), unroll=False)
    pltpu.sync_copy(deg_v.at[pl.ds(0, TPB)],
                    deg_hbm.at[pl.ds(t * jnp.int32(TPB), TPB)])


def _sc_mp_body(h_hbm, slist_hbm, llist_hbm, cnt_hbm, out_hbm,
                si_v, di_v, rows_v, cbuf, acc_v):
    t = _wid()
    lane = lax.iota(jnp.int32, 16)

    def zero(i, _):
        for j in range(D // 16):
            acc_v[i, pl.ds(j * 16, 16)] = jnp.zeros((16,), jnp.float32)
        return jnp.int32(0)

    lax.fori_loop(jnp.int32(0), jnp.int32(TPB + 1), zero, jnp.int32(0),
                  unroll=False)
    pltpu.sync_copy(cnt_hbm.at[t], cbuf)
    total = cbuf[pl.ds(0, 16)][0]
    nch = (total + jnp.int32(K - 1)) // jnp.int32(K)

    def chunk(ci, _):
        base = t * jnp.int32(REG) + jnp.int32(ci) * jnp.int32(K)
        pltpu.sync_copy(slist_hbm.at[pl.ds(base, K)], si_v)
        pltpu.sync_copy(llist_hbm.at[pl.ds(base, K)], di_v)
        gbase = jnp.int32(ci) * jnp.int32(K)

        def fix(v, _):
            o = jnp.int32(v) * 16
            valid = (gbase + o + lane) < total
            sv = jnp.clip(si_v[pl.ds(o, 16)], 0, jnp.int32(N - 1))
            si_v[pl.ds(o, 16)] = jnp.where(valid, sv, jnp.int32(0))
            dv = jnp.clip(di_v[pl.ds(o, 16)], 0, jnp.int32(TPB))
            di_v[pl.ds(o, 16)] = jnp.where(valid, dv, jnp.int32(TPB))
            return jnp.int32(0)

        lax.fori_loop(jnp.int32(0), jnp.int32(K // 16), fix, jnp.int32(0),
                      unroll=False)
        pltpu.sync_copy(h_hbm.at[si_v], rows_v)

        def gbody(g, _):
            o = jnp.int32(g) * 16
            dv = di_v[pl.ds(o, 16)]
            for l in range(16):
                e = o + jnp.int32(l)
                r = dv[l]
                for j in range(D // 16):
                    plsc.addupdate(acc_v.at[r, pl.ds(j * 16, 16)],
                                   rows_v[e, pl.ds(j * 16, 16)])
            return jnp.int32(0)

        lax.fori_loop(jnp.int32(0), jnp.int32(K // 16), gbody, jnp.int32(0),
                      unroll=False)
        return jnp.int32(0)

    lax.fori_loop(jnp.int32(0), nch, chunk, jnp.int32(0), unroll=False)
    pltpu.sync_copy(acc_v.at[pl.ds(0, TPB)],
                    out_hbm.at[pl.ds(t * jnp.int32(TPB), TPB)])


@functools.cache
def _sc_kernels():
    mesh = plsc.VectorSubcoreMesh(core_axis_name="c", subcore_axis_name="s",
                                  num_cores=NC, num_subcores=NS)
    deg = pl.kernel(
        _sc_deg_body,
        out_type=jax.ShapeDtypeStruct((N_ACC, 16), jnp.int32),
        mesh=mesh,
        scratch_types=[
            pltpu.VMEM((SCH,), jnp.int32),
            pltpu.VMEM((16,), jnp.int32),
            pltpu.VMEM((TPB + 1, 16), jnp.int32),
        ],
    )
    mp = pl.kernel(
        _sc_mp_body,
        out_type=jax.ShapeDtypeStruct((N_ACC, D), jnp.float32),
        mesh=mesh,
        scratch_types=[
            pltpu.VMEM((K,), jnp.int32),
            pltpu.VMEM((K,), jnp.int32),
            pltpu.VMEM((K, D), jnp.float32),
            pltpu.VMEM((16,), jnp.int32),
            pltpu.VMEM((TPB + 1, D), jnp.float32),
        ],
    )
    return deg, mp


BR = 1000


def _prep_body(deg_ref, x_ref, w1_ref, h_ref, dinv_ref):
    deg = deg_ref[...].astype(jnp.float32) + 1.0
    dinv = lax.rsqrt(deg)
    h = jnp.dot(x_ref[...], w1_ref[...], preferred_element_type=jnp.float32)
    h_ref[...] = h * dinv
    dinv_ref[...] = dinv


def _mid_body(acc_ref, h1_ref, dinv_ref, b1_ref, w2_ref, out_ref):
    conv = (acc_ref[...] + h1_ref[...]) * dinv_ref[...] + b1_ref[...]
    h2 = jnp.maximum(conv, 0.0)
    out_ref[...] = jnp.dot(h2, w2_ref[...],
                           preferred_element_type=jnp.float32) * dinv_ref[...]


def _fin_body(acc_ref, h2_ref, dinv_ref, b2_ref, wfc_ref, bfc_ref, out_ref):
    conv = (acc_ref[...] + h2_ref[...]) * dinv_ref[...] + b2_ref[...]
    h3 = jnp.maximum(conv, 0.0)
    logits = jnp.dot(h3, wfc_ref[...],
                     preferred_element_type=jnp.float32) + bfc_ref[...]
    m = jnp.max(logits, axis=1, keepdims=True)
    sh = logits - m
    out_ref[...] = sh - jnp.log(jnp.sum(jnp.exp(sh), axis=1, keepdims=True))


_GRID = N // BR

_prep = pl.pallas_call(
    _prep_body,
    grid=(_GRID,),
    in_specs=[
        pl.BlockSpec((BR, 1), lambda i: (i, jnp.int32(0))),
        pl.BlockSpec((BR, D), lambda i: (i, jnp.int32(0))),
        pl.BlockSpec((D, D), lambda i: (jnp.int32(0), jnp.int32(0))),
    ],
    out_specs=[
        pl.BlockSpec((BR, D), lambda i: (i, jnp.int32(0))),
        pl.BlockSpec((BR, 1), lambda i: (i, jnp.int32(0))),
    ],
    out_shape=[
        jax.ShapeDtypeStruct((N, D), jnp.float32),
        jax.ShapeDtypeStruct((N, 1), jnp.float32),
    ],
)

_mid = pl.pallas_call(
    _mid_body,
    grid=(_GRID,),
    in_specs=[
        pl.BlockSpec((BR, D), lambda i: (i, jnp.int32(0))),
        pl.BlockSpec((BR, D), lambda i: (i, jnp.int32(0))),
        pl.BlockSpec((BR, 1), lambda i: (i, jnp.int32(0))),
        pl.BlockSpec((1, D), lambda i: (jnp.int32(0), jnp.int32(0))),
        pl.BlockSpec((D, D), lambda i: (jnp.int32(0), jnp.int32(0))),
    ],
    out_specs=pl.BlockSpec((BR, D), lambda i: (i, jnp.int32(0))),
    out_shape=jax.ShapeDtypeStruct((N, D), jnp.float32),
)

_fin = pl.pallas_call(
    _fin_body,
    grid=(_GRID,),
    in_specs=[
        pl.BlockSpec((BR, D), lambda i: (i, jnp.int32(0))),
        pl.BlockSpec((BR, D), lambda i: (i, jnp.int32(0))),
        pl.BlockSpec((BR, 1), lambda i: (i, jnp.int32(0))),
        pl.BlockSpec((1, D), lambda i: (jnp.int32(0), jnp.int32(0))),
        pl.BlockSpec((D, 2), lambda i: (jnp.int32(0), jnp.int32(0))),
        pl.BlockSpec((1, 2), lambda i: (jnp.int32(0), jnp.int32(0))),
    ],
    out_specs=pl.BlockSpec((BR, 2), lambda i: (i, jnp.int32(0))),
    out_shape=jax.ShapeDtypeStruct((N, 2), jnp.float32),
)


def kernel(x, edge_index, W1, b1, W2, b2, Wfc, bfc):
    src = edge_index[0].astype(jnp.int32)
    dst = edge_index[1].astype(jnp.int32)
    pad = E_PAD - E
    src_p = jnp.concatenate([src, jnp.zeros((pad,), jnp.int32)])
    dst_p = jnp.concatenate([dst, jnp.full((pad,), N, jnp.int32)])

    x = x.astype(jnp.float32)
    degk, mp = _sc_kernels()
    owner = dst_p // TPB
    order = jnp.argsort(owner, stable=True)
    owner_s = owner[order]
    counts = jnp.bincount(owner, length=NW).astype(jnp.int32)
    starts = jnp.concatenate([jnp.zeros((1,), jnp.int32),
                              jnp.cumsum(counts)[:-1].astype(jnp.int32)])
    rank = jnp.arange(E_PAD, dtype=jnp.int32) - starts[owner_s]
    pos = owner_s.astype(jnp.int32) * REG + rank
    slist = jnp.zeros((NW * REG,), jnp.int32).at[pos].set(src_p[order])
    llist = jnp.zeros((NW * REG,), jnp.int32).at[pos].set(
        (dst_p - owner * TPB)[order])
    cnts = jnp.broadcast_to(counts[:, None], (NW, 16)).astype(jnp.int32)
    deg = degk(llist, cnts)[:N, 0:1]
    h1, dinv = _prep(deg, x, W1.astype(jnp.float32))
    acc1 = mp(h1, slist, llist, cnts)[:N]
    h2 = _mid(acc1, h1, dinv, b1.astype(jnp.float32).reshape(1, D),
              W2.astype(jnp.float32))
    acc2 = mp(h2, slist, llist, cnts)[:N]
    out = _fin(acc2, h2, dinv, b2.astype(jnp.float32).reshape(1, D),
               Wfc.astype(jnp.float32), bfc.astype(jnp.float32).reshape(1, 2))
    return out
